# baseline (speedup 1.0000x reference)
import functools

import jax
import jax.numpy as jnp
from jax import lax
from jax.experimental import pallas as pl
from jax.experimental.pallas import tpu as pltpu
from jax.experimental.pallas import tpu_sc as plsc

NUM_CORES = 2
NUM_SUBCORES = 16
NUM_WORKERS = NUM_CORES * NUM_SUBCORES


@functools.lru_cache(maxsize=None)
def _build_gather(n_rows: int, vocab: int, dim: int, chunk: int):
    assert n_rows % NUM_WORKERS == 0
    per_w = n_rows // NUM_WORKERS
    assert per_w % chunk == 0
    n_chunks = per_w // chunk
    mesh = plsc.VectorSubcoreMesh(core_axis_name="c", subcore_axis_name="s")

    @functools.partial(
        pl.kernel,
        mesh=mesh,
        compiler_params=pltpu.CompilerParams(use_tc_tiling_on_sc=False),
        out_type=jax.ShapeDtypeStruct((n_rows, dim), jnp.float32),
        scratch_types=[
            pltpu.VMEM((chunk,), jnp.int32),
            pltpu.VMEM((chunk, dim), jnp.float32),
            pltpu.SemaphoreType.DMA,
        ],
    )
    def gather_kernel(table_hbm, idx_hbm, out_hbm, idx_v, rows_v, sem):
        wid = lax.axis_index("s") * NUM_CORES + lax.axis_index("c")
        base = wid * per_w
        for c in range(n_chunks):
            off = base + c * chunk
            pltpu.sync_copy(idx_hbm.at[pl.ds(off, chunk)], idx_v)
            pltpu.async_copy(table_hbm.at[idx_v], rows_v, sem).wait()
            pltpu.sync_copy(rows_v, out_hbm.at[pl.ds(off, chunk)])

    return gather_kernel


def kernel(inputs, embed_weight):
    batch, fields = inputs.shape
    vocab, dim = embed_weight.shape
    flat_idx = inputs.reshape(-1).astype(jnp.int32)
    n_rows = batch * fields
    gather = _build_gather(n_rows, vocab, dim, chunk=1664)
    out = gather(embed_weight, flat_idx)
    return out.reshape(batch, fields, dim)


# speedup vs baseline: 1.0114x; 1.0114x over previous
import functools

import jax
import jax.numpy as jnp
from jax import lax
from jax.experimental import pallas as pl
from jax.experimental.pallas import tpu as pltpu
from jax.experimental.pallas import tpu_sc as plsc

NUM_CORES = 2
NUM_SUBCORES = 16
NUM_WORKERS = NUM_CORES * NUM_SUBCORES
NBUF = 2


@functools.lru_cache(maxsize=None)
def _build_gather(n_rows: int, vocab: int, dim: int, chunk: int):
    assert n_rows % NUM_WORKERS == 0
    per_w = n_rows // NUM_WORKERS
    assert per_w % chunk == 0
    n_chunks = per_w // chunk
    mesh = plsc.VectorSubcoreMesh(core_axis_name="c", subcore_axis_name="s")

    @functools.partial(
        pl.kernel,
        mesh=mesh,
        compiler_params=pltpu.CompilerParams(use_tc_tiling_on_sc=False),
        out_type=jax.ShapeDtypeStruct((n_rows, dim), jnp.float32),
        scratch_types=[
            pltpu.VMEM((per_w,), jnp.int32),
            pltpu.VMEM((NBUF, chunk, dim), jnp.float32),
            pltpu.SemaphoreType.DMA((NBUF,)),
            pltpu.SemaphoreType.DMA((NBUF,)),
        ],
    )
    def gather_kernel(table_hbm, idx_hbm, out_hbm, idx_v, rows_v, gsem, ssem):
        wid = lax.axis_index("s") * NUM_CORES + lax.axis_index("c")
        base = wid * per_w
        pltpu.sync_copy(idx_hbm.at[pl.ds(base, per_w)], idx_v)

        def gather_start(c):
            b = c % NBUF
            pltpu.async_copy(
                table_hbm.at[idx_v.at[pl.ds(c * chunk, chunk)]],
                rows_v.at[b], gsem.at[b])

        def store_start(c):
            b = c % NBUF
            pltpu.async_copy(
                rows_v.at[b], out_hbm.at[pl.ds(base + c * chunk, chunk)],
                ssem.at[b])

        gather_start(0)
        for c in range(n_chunks):
            b = c % NBUF
            if c + 1 < n_chunks:
                if c + 1 >= NBUF:
                    pltpu.make_async_copy(
                        rows_v.at[(c + 1) % NBUF],
                        out_hbm.at[pl.ds(0, chunk)],
                        ssem.at[(c + 1) % NBUF]).wait()
                gather_start(c + 1)
            pltpu.make_async_copy(
                table_hbm.at[idx_v.at[pl.ds(0, chunk)]],
                rows_v.at[b], gsem.at[b]).wait()
            store_start(c)
        for c in range(max(0, n_chunks - NBUF + 1), n_chunks):
            b = c % NBUF
            pltpu.make_async_copy(
                rows_v.at[b], out_hbm.at[pl.ds(0, chunk)], ssem.at[b]).wait()

    return gather_kernel


def kernel(inputs, embed_weight):
    batch, fields = inputs.shape
    vocab, dim = embed_weight.shape
    flat_idx = inputs.reshape(-1).astype(jnp.int32)
    n_rows = batch * fields
    gather = _build_gather(n_rows, vocab, dim, chunk=1664)
    out = gather(embed_weight, flat_idx)
    return out.reshape(batch, fields, dim)


# speedup vs baseline: 1.7632x; 1.7434x over previous
import functools

import jax
import jax.numpy as jnp
from jax import lax
from jax.experimental import pallas as pl
from jax.experimental.pallas import tpu as pltpu
from jax.experimental.pallas import tpu_sc as plsc

NUM_CORES = 2
NUM_SUBCORES = 16
NUM_WORKERS = NUM_CORES * NUM_SUBCORES
BLOCK = 128
LANES = 16


@functools.lru_cache(maxsize=None)
def _build_gather(batch: int, fields: int, vocab: int, dim: int):
    per_w = batch // NUM_WORKERS
    n_blocks = per_w // BLOCK
    assert batch % NUM_WORKERS == 0 and per_w % BLOCK == 0 and dim == LANES
    groups = BLOCK // LANES
    mesh = plsc.VectorSubcoreMesh(core_axis_name="c", subcore_axis_name="s")

    @functools.partial(
        pl.kernel,
        mesh=mesh,
        compiler_params=pltpu.CompilerParams(use_tc_tiling_on_sc=False),
        out_type=jax.ShapeDtypeStruct((fields * dim, batch), jnp.float32),
        scratch_types=[
            pltpu.VMEM((fields, per_w), jnp.int32),
            pltpu.VMEM((fields * BLOCK, dim), jnp.float32),
            pltpu.VMEM((fields * dim, BLOCK), jnp.float32),
            pltpu.SemaphoreType.DMA,
        ],
    )
    def gather_kernel(table_hbm, idx_hbm, out_hbm, idx_v, rows_v, slab_v, sem):
        wid = lax.axis_index("s") * NUM_CORES + lax.axis_index("c")
        base = wid * per_w
        pltpu.sync_copy(idx_hbm.at[:, pl.ds(base, per_w)], idx_v)
        lanes = lax.iota(jnp.int32, LANES)
        perms = [lanes ^ s for s in (1, 2, 4, 8)]
        masks = [(lanes & s) == 0 for s in (1, 2, 4, 8)]

        def transpose16(vv):
            for si, s in enumerate((1, 2, 4, 8)):
                nv = []
                for i in range(LANES):
                    p = vv[i ^ s].at[perms[si]].get(mode="promise_in_bounds")
                    if i & s == 0:
                        nv.append(jnp.where(masks[si], vv[i], p))
                    else:
                        nv.append(jnp.where(masks[si], p, vv[i]))
                vv = nv
            return vv

        for k in range(n_blocks):
            for f in range(fields):
                pltpu.async_copy(
                    table_hbm.at[idx_v.at[f, pl.ds(k * BLOCK, BLOCK)]],
                    rows_v.at[pl.ds(f * BLOCK, BLOCK)], sem)
            for f in range(fields):
                pltpu.make_async_copy(
                    table_hbm.at[idx_v.at[f, pl.ds(k * BLOCK, BLOCK)]],
                    rows_v.at[pl.ds(f * BLOCK, BLOCK)], sem).wait()

            def tp_body(t, _):
                f = t // groups
                g = t % groups
                rbase = f * BLOCK + g * LANES
                vv = [rows_v[rbase + r, :] for r in range(LANES)]
                vv = transpose16(vv)
                cbase = f * dim
                for d in range(LANES):
                    slab_v[cbase + d, pl.ds(g * LANES, LANES)] = vv[d]
                return _

            lax.fori_loop(0, fields * groups, tp_body, None)
            pltpu.sync_copy(
                slab_v, out_hbm.at[:, pl.ds(base + k * BLOCK, BLOCK)])

    return gather_kernel


def kernel(inputs, embed_weight):
    batch, fields = inputs.shape
    vocab, dim = embed_weight.shape
    idx_t = inputs.T.astype(jnp.int32)
    gather = _build_gather(batch, fields, vocab, dim)
    out = gather(embed_weight, idx_t)
    return out.reshape(fields, dim, batch).transpose(2, 0, 1)


# speedup vs baseline: 1.9292x; 1.0941x over previous
import functools

import jax
import jax.numpy as jnp
from jax import lax
from jax.experimental import pallas as pl
from jax.experimental.pallas import tpu as pltpu
from jax.experimental.pallas import tpu_sc as plsc

NUM_CORES = 2
NUM_SUBCORES = 16
NUM_WORKERS = NUM_CORES * NUM_SUBCORES
BLOCK = 64
LANES = 16


@functools.lru_cache(maxsize=None)
def _build_gather(batch: int, fields: int, vocab: int, dim: int):
    per_w = batch // NUM_WORKERS
    n_blocks = per_w // BLOCK
    assert batch % NUM_WORKERS == 0 and per_w % BLOCK == 0 and dim == LANES
    groups = BLOCK // LANES
    n_tcols = batch // 128
    mesh = plsc.VectorSubcoreMesh(core_axis_name="c", subcore_axis_name="s")

    @functools.partial(
        pl.kernel,
        mesh=mesh,
        compiler_params=pltpu.CompilerParams(use_tc_tiling_on_sc=False),
        out_type=jax.ShapeDtypeStruct((fields, dim // 8, n_tcols, 8, 128),
                                      jnp.float32),
        scratch_types=[
            pltpu.VMEM((fields, per_w), jnp.int32),
            pltpu.VMEM((2, fields * BLOCK, dim), jnp.float32),
            pltpu.VMEM((2, fields, dim // 8, 8, BLOCK), jnp.float32),
            pltpu.SemaphoreType.DMA((2,)),
            pltpu.SemaphoreType.DMA((2,)),
        ],
    )
    def gather_kernel(table_hbm, idx_hbm, out_hbm, idx_v, rows_v, slab_v,
                      gsem, osem):
        wid = lax.axis_index("s") * NUM_CORES + lax.axis_index("c")
        base = wid * per_w
        pltpu.sync_copy(idx_hbm.at[:, pl.ds(base, per_w)], idx_v)
        lanes = lax.iota(jnp.int32, LANES)
        perms = [lanes ^ s for s in (1, 2, 4, 8)]
        masks = [(lanes & s) == 0 for s in (1, 2, 4, 8)]

        def transpose16(vv):
            for si, s in enumerate((1, 2, 4, 8)):
                nv = []
                for i in range(LANES):
                    p = vv[i ^ s].at[perms[si]].get(mode="promise_in_bounds")
                    if i & s == 0:
                        nv.append(jnp.where(masks[si], vv[i], p))
                    else:
                        nv.append(jnp.where(masks[si], p, vv[i]))
                vv = nv
            return vv

        def gather_start(k):
            b = k % 2
            for f in range(fields):
                pltpu.async_copy(
                    table_hbm.at[idx_v.at[f, pl.ds(k * BLOCK, BLOCK)]],
                    rows_v.at[b, pl.ds(f * BLOCK, BLOCK)], gsem.at[b])

        def gather_drain(k):
            b = k % 2
            for f in range(fields):
                pltpu.make_async_copy(
                    table_hbm.at[idx_v.at[f, pl.ds(k * BLOCK, BLOCK)]],
                    rows_v.at[b, pl.ds(f * BLOCK, BLOCK)], gsem.at[b]).wait()

        def out_view(k):
            b0 = base + k * BLOCK
            return out_hbm.at[:, :, b0 // 128, :, pl.ds(b0 % 128, BLOCK)]

        def out_start(k):
            b = k % 2
            pltpu.async_copy(slab_v.at[b], out_view(k), osem.at[b])

        def out_drain(k):
            b = k % 2
            pltpu.make_async_copy(slab_v.at[b], out_view(k), osem.at[b]).wait()

        def transpose_block(k):
            b = k % 2

            def tp_body(t, _):
                f = t // groups
                g = t % groups
                rbase = f * BLOCK + g * LANES
                vv = [rows_v[b, rbase + r, :] for r in range(LANES)]
                vv = transpose16(vv)
                for d in range(LANES):
                    slab_v[b, f, d // 8, d % 8, pl.ds(g * LANES, LANES)] = vv[d]
                return _

            lax.fori_loop(0, fields * groups, tp_body, None)

        gather_start(0)
        for k in range(n_blocks):
            if k + 1 < n_blocks:
                gather_start(k + 1)
            gather_drain(k)
            if k >= 2:
                out_drain(k - 2)
            transpose_block(k)
            out_start(k)
        for k in range(max(0, n_blocks - 2), n_blocks):
            out_drain(k)

    return gather_kernel


def kernel(inputs, embed_weight):
    batch, fields = inputs.shape
    vocab, dim = embed_weight.shape
    idx_t = inputs.T.astype(jnp.int32)
    gather = _build_gather(batch, fields, vocab, dim)
    out5 = gather(embed_weight, idx_t)
    out = out5.transpose(2, 4, 0, 1, 3).reshape(batch, fields, dim)
    return out


# speedup vs baseline: 2.9213x; 1.5143x over previous
import functools

import jax
import jax.numpy as jnp
from jax import lax
from jax.experimental import pallas as pl
from jax.experimental.pallas import tpu as pltpu
from jax.experimental.pallas import tpu_sc as plsc

NUM_CORES = 2
NUM_SUBCORES = 16
NUM_WORKERS = NUM_CORES * NUM_SUBCORES
BLOCK = 64
LANES = 16


@functools.lru_cache(maxsize=None)
def _build_format(vocab: int, dim: int):
    full_tcols = vocab // 128
    rem = vocab - full_tcols * 128
    rem_rows = rem * dim // 128
    mesh = plsc.VectorSubcoreMesh(core_axis_name="c", subcore_axis_name="s")
    per_w = (full_tcols + NUM_WORKERS - 1) // NUM_WORKERS

    @functools.partial(
        pl.kernel,
        mesh=mesh,
        compiler_params=pltpu.CompilerParams(use_tc_tiling_on_sc=True),
        out_type=jax.ShapeDtypeStruct((vocab * dim // 128, 128), jnp.float32),
        scratch_types=[
            pltpu.VMEM((LANES, 128), jnp.float32),
            pltpu.VMEM((LANES, 128), jnp.float32),
        ],
    )
    def format_kernel(table_hbm, rem_hbm, out_hbm, vbuf, slab):
        wid = lax.axis_index("s") * NUM_CORES + lax.axis_index("c")
        lanes = lax.iota(jnp.int32, LANES)
        perms = [lanes ^ s for s in (1, 2, 4, 8)]
        masks = [(lanes & s) == 0 for s in (1, 2, 4, 8)]

        def transpose16(vv):
            for si, s in enumerate((1, 2, 4, 8)):
                nv = []
                for i in range(LANES):
                    p = vv[i ^ s].at[perms[si]].get(mode="promise_in_bounds")
                    if i & s == 0:
                        nv.append(jnp.where(masks[si], vv[i], p))
                    else:
                        nv.append(jnp.where(masks[si], p, vv[i]))
                vv = nv
            return vv

        def do_col(c):
            pltpu.sync_copy(table_hbm.at[:, pl.ds(c * 128, 128)], vbuf)
            for g in range(8):
                vv = [vbuf[r, pl.ds(g * LANES, LANES)] for r in range(LANES)]
                vv = transpose16(vv)
                for j in range(8):
                    slab[2 * g, pl.ds(j * LANES, LANES)] = vv[j]
                    slab[2 * g + 1, pl.ds(j * LANES, LANES)] = vv[8 + j]
            pltpu.sync_copy(slab, out_hbm.at[pl.ds(c * dim, dim), :])

        def body(k, _):
            c = wid + k * NUM_WORKERS

            @pl.when(c < full_tcols)
            def _full():
                do_col(c)
            return _

        lax.fori_loop(0, per_w, body, None)

        if rem:
            @pl.when(wid == NUM_WORKERS - 1)
            def _tail():
                pltpu.sync_copy(rem_hbm, vbuf.at[pl.ds(0, rem_rows), :])
                pltpu.sync_copy(vbuf.at[pl.ds(0, rem_rows), :],
                                out_hbm.at[pl.ds(full_tcols * dim, rem_rows), :])

    return format_kernel


@functools.lru_cache(maxsize=None)
def _build_gather(batch: int, fields: int, vocab: int, dim: int):
    per_w = batch // NUM_WORKERS
    n_blocks = per_w // BLOCK
    assert batch % NUM_WORKERS == 0 and per_w % BLOCK == 0 and dim == LANES
    groups = BLOCK // LANES
    n_tcols = batch // 128
    mesh = plsc.VectorSubcoreMesh(core_axis_name="c", subcore_axis_name="s")

    @functools.partial(
        pl.kernel,
        mesh=mesh,
        compiler_params=pltpu.CompilerParams(use_tc_tiling_on_sc=False),
        out_type=jax.ShapeDtypeStruct((fields, dim // 8, n_tcols, 8, 128),
                                      jnp.float32),
        scratch_types=[
            pltpu.VMEM((fields, per_w), jnp.int32),
            pltpu.VMEM((2, fields * BLOCK, dim), jnp.float32),
            pltpu.VMEM((2, fields, dim // 8, 8, BLOCK), jnp.float32),
            pltpu.SemaphoreType.DMA((2,)),
            pltpu.SemaphoreType.DMA((2,)),
        ],
    )
    def gather_kernel(table_hbm, idx_hbm, out_hbm, idx_v, rows_v, slab_v,
                      gsem, osem):
        wid = lax.axis_index("s") * NUM_CORES + lax.axis_index("c")
        base = wid * per_w
        pltpu.sync_copy(idx_hbm.at[:, pl.ds(base, per_w)], idx_v)
        lanes = lax.iota(jnp.int32, LANES)
        perms = [lanes ^ s for s in (1, 2, 4, 8)]
        masks = [(lanes & s) == 0 for s in (1, 2, 4, 8)]

        def transpose16(vv):
            for si, s in enumerate((1, 2, 4, 8)):
                nv = []
                for i in range(LANES):
                    p = vv[i ^ s].at[perms[si]].get(mode="promise_in_bounds")
                    if i & s == 0:
                        nv.append(jnp.where(masks[si], vv[i], p))
                    else:
                        nv.append(jnp.where(masks[si], p, vv[i]))
                vv = nv
            return vv

        def gather_start(k):
            b = k % 2
            for f in range(fields):
                pltpu.async_copy(
                    table_hbm.at[idx_v.at[f, pl.ds(k * BLOCK, BLOCK)]],
                    rows_v.at[b, pl.ds(f * BLOCK, BLOCK)], gsem.at[b])

        def gather_drain(k):
            b = k % 2
            for f in range(fields):
                pltpu.make_async_copy(
                    table_hbm.at[idx_v.at[f, pl.ds(k * BLOCK, BLOCK)]],
                    rows_v.at[b, pl.ds(f * BLOCK, BLOCK)], gsem.at[b]).wait()

        def out_view(k):
            b0 = base + k * BLOCK
            return out_hbm.at[:, :, b0 // 128, :, pl.ds(b0 % 128, BLOCK)]

        def out_start(k):
            b = k % 2
            pltpu.async_copy(slab_v.at[b], out_view(k), osem.at[b])

        def out_drain(k):
            b = k % 2
            pltpu.make_async_copy(slab_v.at[b], out_view(k), osem.at[b]).wait()

        def transpose_block(k):
            b = k % 2

            def tp_body(t, _):
                f = t // groups
                g = t % groups
                rbase = f * BLOCK + g * LANES
                vv = [rows_v[b, rbase + r, :] for r in range(LANES)]
                vv = transpose16(vv)
                for d in range(LANES):
                    slab_v[b, f, d // 8, d % 8, pl.ds(g * LANES, LANES)] = vv[d]
                return _

            lax.fori_loop(0, fields * groups, tp_body, None)

        gather_start(0)
        for k in range(n_blocks):
            if k + 1 < n_blocks:
                gather_start(k + 1)
            gather_drain(k)
            if k >= 2:
                out_drain(k - 2)
            transpose_block(k)
            out_start(k)
        for k in range(max(0, n_blocks - 2), n_blocks):
            out_drain(k)

    return gather_kernel


def kernel(inputs, embed_weight):
    batch, fields = inputs.shape
    vocab, dim = embed_weight.shape
    idx_t = inputs.T.astype(jnp.int32)
    fmt = _build_format(vocab, dim)
    full_tcols = vocab // 128
    rem8 = embed_weight[full_tcols * 128:, :].reshape(-1, 128)
    table_lin = fmt(embed_weight.T, rem8).reshape(vocab, dim)
    gather = _build_gather(batch, fields, vocab, dim)
    out5 = gather(table_lin, idx_t)
    out = out5.transpose(2, 4, 0, 1, 3).reshape(batch, fields, dim)
    return out


# speedup vs baseline: 3.9326x; 1.3462x over previous
import functools

import jax
import jax.numpy as jnp
from jax import lax
from jax.experimental import pallas as pl
from jax.experimental.pallas import tpu as pltpu
from jax.experimental.pallas import tpu_sc as plsc

NUM_CORES = 2
NUM_SUBCORES = 16
NUM_WORKERS = NUM_CORES * NUM_SUBCORES
BLOCK = 64
LANES = 16


@functools.lru_cache(maxsize=None)
def _build_format(vocab: int, dim: int):
    full_tcols = vocab // 128
    rem = vocab - full_tcols * 128
    rem_rows = rem * dim // 128
    mesh = plsc.VectorSubcoreMesh(core_axis_name="c", subcore_axis_name="s")
    base_w = full_tcols // NUM_WORKERS
    n_extra = full_tcols - base_w * NUM_WORKERS
    CH = 2
    n_chunks = base_w // CH
    assert base_w % CH == 0

    @functools.partial(
        pl.kernel,
        mesh=mesh,
        compiler_params=pltpu.CompilerParams(use_tc_tiling_on_sc=True),
        out_type=jax.ShapeDtypeStruct((vocab * dim // 128, 128), jnp.float32),
        scratch_types=[
            pltpu.VMEM((2, LANES, CH * 128), jnp.float32),
            pltpu.VMEM((2, CH * LANES, 128), jnp.float32),
            pltpu.SemaphoreType.DMA((2,)),
            pltpu.SemaphoreType.DMA((2,)),
        ],
    )
    def format_kernel(table_hbm, rem_hbm, out_hbm, vbuf, slab, isem, osem):
        wid = lax.axis_index("s") * NUM_CORES + lax.axis_index("c")
        start = wid * base_w + jnp.minimum(wid, n_extra)
        lanes = lax.iota(jnp.int32, LANES)
        perms = [lanes ^ s for s in (1, 2, 4, 8)]
        masks = [(lanes & s) == 0 for s in (1, 2, 4, 8)]

        def transpose16(vv):
            for si, s in enumerate((1, 2, 4, 8)):
                nv = []
                for i in range(LANES):
                    p = vv[i ^ s].at[perms[si]].get(mode="promise_in_bounds")
                    if i & s == 0:
                        nv.append(jnp.where(masks[si], vv[i], p))
                    else:
                        nv.append(jnp.where(masks[si], p, vv[i]))
                vv = nv
            return vv

        def in_view(k):
            return table_hbm.at[:, pl.ds((start + k * CH) * 128, CH * 128)]

        def out_rows(k):
            return out_hbm.at[pl.ds((start + k * CH) * dim, CH * dim), :]

        def phase_full(k, b):
            pltpu.make_async_copy(in_view(k), vbuf.at[b], isem.at[b]).wait()

            @pl.when(k >= 2)
            def _drain_out():
                pltpu.make_async_copy(slab.at[b], out_rows(k - 2),
                                      osem.at[b]).wait()

            for cc in range(CH):
                for g in range(8):
                    vv = [vbuf[b, r, pl.ds(cc * 128 + g * LANES, LANES)]
                          for r in range(LANES)]
                    vv = transpose16(vv)
                    for j in range(8):
                        slab[b, cc * LANES + 2 * g, pl.ds(j * LANES, LANES)] = vv[j]
                        slab[b, cc * LANES + 2 * g + 1, pl.ds(j * LANES, LANES)] = vv[8 + j]
            pltpu.async_copy(slab.at[b], out_rows(k), osem.at[b])

            @pl.when(k + 2 < n_chunks)
            def _next_in():
                pltpu.async_copy(in_view(k + 2), vbuf.at[b], isem.at[b])

        pltpu.async_copy(in_view(0), vbuf.at[0], isem.at[0])
        pltpu.async_copy(in_view(1), vbuf.at[1], isem.at[1])

        def body(k, _):
            @pl.when(k % 2 == 0)
            def _even():
                phase_full(k, 0)

            @pl.when(k % 2 == 1)
            def _odd():
                phase_full(k, 1)
            return _

        lax.fori_loop(0, n_chunks, body, None)
        for b, k in ((0, n_chunks - 2), (1, n_chunks - 1)):
            pltpu.make_async_copy(slab.at[b], out_rows(k), osem.at[b]).wait()

        @pl.when(wid < n_extra)
        def _extra():
            c = start + base_w
            pltpu.sync_copy(table_hbm.at[:, pl.ds(c * 128, 128)],
                            vbuf.at[0, :, pl.ds(0, 128)])
            for g in range(8):
                vv = [vbuf[0, r, pl.ds(g * LANES, LANES)]
                      for r in range(LANES)]
                vv = transpose16(vv)
                for j in range(8):
                    slab[0, 2 * g, pl.ds(j * LANES, LANES)] = vv[j]
                    slab[0, 2 * g + 1, pl.ds(j * LANES, LANES)] = vv[8 + j]
            pltpu.sync_copy(slab.at[0, pl.ds(0, dim), :],
                            out_hbm.at[pl.ds(c * dim, dim), :])

        if rem:
            @pl.when(wid == NUM_WORKERS - 1)
            def _tail():
                pltpu.sync_copy(rem_hbm,
                                vbuf.at[0, pl.ds(0, rem_rows), pl.ds(0, 128)])
                pltpu.sync_copy(vbuf.at[0, pl.ds(0, rem_rows), pl.ds(0, 128)],
                                out_hbm.at[pl.ds(full_tcols * dim, rem_rows), :])

    return format_kernel


@functools.lru_cache(maxsize=None)
def _build_gather(batch: int, fields: int, vocab: int, dim: int):
    per_w = batch // NUM_WORKERS
    n_blocks = per_w // BLOCK
    assert batch % NUM_WORKERS == 0 and per_w % BLOCK == 0 and dim == LANES
    groups = BLOCK // LANES
    n_tcols = batch // 128
    mesh = plsc.VectorSubcoreMesh(core_axis_name="c", subcore_axis_name="s")

    @functools.partial(
        pl.kernel,
        mesh=mesh,
        compiler_params=pltpu.CompilerParams(use_tc_tiling_on_sc=False),
        out_type=jax.ShapeDtypeStruct((fields, dim // 8, n_tcols, 8, 128),
                                      jnp.float32),
        scratch_types=[
            pltpu.VMEM((fields, per_w), jnp.int32),
            pltpu.VMEM((2, fields * BLOCK, dim), jnp.float32),
            pltpu.VMEM((2, fields, dim // 8, 8, BLOCK), jnp.float32),
            pltpu.SemaphoreType.DMA((2,)),
            pltpu.SemaphoreType.DMA((2,)),
        ],
    )
    def gather_kernel(table_hbm, idx_hbm, out_hbm, idx_v, rows_v, slab_v,
                      gsem, osem):
        wid = lax.axis_index("s") * NUM_CORES + lax.axis_index("c")
        base = wid * per_w
        pltpu.sync_copy(idx_hbm.at[:, pl.ds(base, per_w)], idx_v)
        lanes = lax.iota(jnp.int32, LANES)
        perms = [lanes ^ s for s in (1, 2, 4, 8)]
        masks = [(lanes & s) == 0 for s in (1, 2, 4, 8)]

        def transpose16(vv):
            for si, s in enumerate((1, 2, 4, 8)):
                nv = []
                for i in range(LANES):
                    p = vv[i ^ s].at[perms[si]].get(mode="promise_in_bounds")
                    if i & s == 0:
                        nv.append(jnp.where(masks[si], vv[i], p))
                    else:
                        nv.append(jnp.where(masks[si], p, vv[i]))
                vv = nv
            return vv

        def gather_start(k):
            b = k % 2
            for f in range(fields):
                pltpu.async_copy(
                    table_hbm.at[idx_v.at[f, pl.ds(k * BLOCK, BLOCK)]],
                    rows_v.at[b, pl.ds(f * BLOCK, BLOCK)], gsem.at[b])

        def gather_drain(k):
            b = k % 2
            for f in range(fields):
                pltpu.make_async_copy(
                    table_hbm.at[idx_v.at[f, pl.ds(k * BLOCK, BLOCK)]],
                    rows_v.at[b, pl.ds(f * BLOCK, BLOCK)], gsem.at[b]).wait()

        def out_view(k):
            b0 = base + k * BLOCK
            return out_hbm.at[:, :, b0 // 128, :, pl.ds(b0 % 128, BLOCK)]

        def out_start(k):
            b = k % 2
            pltpu.async_copy(slab_v.at[b], out_view(k), osem.at[b])

        def out_drain(k):
            b = k % 2
            pltpu.make_async_copy(slab_v.at[b], out_view(k), osem.at[b]).wait()

        def transpose_block(k):
            b = k % 2

            def tp_body(t, _):
                f = t // groups
                g = t % groups
                rbase = f * BLOCK + g * LANES
                vv = [rows_v[b, rbase + r, :] for r in range(LANES)]
                vv = transpose16(vv)
                for d in range(LANES):
                    slab_v[b, f, d // 8, d % 8, pl.ds(g * LANES, LANES)] = vv[d]
                return _

            lax.fori_loop(0, fields * groups, tp_body, None)

        gather_start(0)
        for k in range(n_blocks):
            if k + 1 < n_blocks:
                gather_start(k + 1)
            gather_drain(k)
            if k >= 2:
                out_drain(k - 2)
            transpose_block(k)
            out_start(k)
        for k in range(max(0, n_blocks - 2), n_blocks):
            out_drain(k)

    return gather_kernel


def kernel(inputs, embed_weight):
    batch, fields = inputs.shape
    vocab, dim = embed_weight.shape
    idx_t = inputs.T.astype(jnp.int32)
    fmt = _build_format(vocab, dim)
    full_tcols = vocab // 128
    rem8 = embed_weight[full_tcols * 128:, :].reshape(-1, 128)
    table_lin = fmt(embed_weight.T, rem8).reshape(vocab, dim)
    gather = _build_gather(batch, fields, vocab, dim)
    out5 = gather(table_lin, idx_t)
    out = out5.transpose(2, 4, 0, 1, 3).reshape(batch, fields, dim)
    return out


# speedup vs baseline: 4.1082x; 1.0446x over previous
import functools

import jax
import jax.numpy as jnp
from jax import lax
from jax.experimental import pallas as pl
from jax.experimental.pallas import tpu as pltpu
from jax.experimental.pallas import tpu_sc as plsc

NUM_CORES = 2
NUM_SUBCORES = 16
NUM_WORKERS = NUM_CORES * NUM_SUBCORES
BLOCK = 64
LANES = 16
NBUF_F = 4


@functools.lru_cache(maxsize=None)
def _build_format(vocab: int, dim: int):
    full_tcols = vocab // 128
    rem = vocab - full_tcols * 128
    rem_rows = rem * dim // 128
    mesh = plsc.VectorSubcoreMesh(core_axis_name="c", subcore_axis_name="s")
    base_w = full_tcols // NUM_WORKERS
    n_extra = full_tcols - base_w * NUM_WORKERS
    CH = 2
    n_chunks = base_w // CH
    assert base_w % CH == 0

    @functools.partial(
        pl.kernel,
        mesh=mesh,
        compiler_params=pltpu.CompilerParams(use_tc_tiling_on_sc=True),
        out_type=jax.ShapeDtypeStruct((vocab * dim // 128, 128), jnp.float32),
        scratch_types=[
            pltpu.VMEM((NBUF_F, LANES, CH * 128), jnp.float32),
            pltpu.VMEM((NBUF_F, CH * LANES, 128), jnp.float32),
            pltpu.SemaphoreType.DMA((NBUF_F,)),
            pltpu.SemaphoreType.DMA((NBUF_F,)),
        ],
    )
    def format_kernel(table_hbm, rem_hbm, out_hbm, vbuf, slab, isem, osem):
        wid = lax.axis_index("s") * NUM_CORES + lax.axis_index("c")
        start = wid * base_w + jnp.minimum(wid, n_extra)
        lanes = lax.iota(jnp.int32, LANES)
        perms = [lanes ^ s for s in (1, 2, 4, 8)]
        masks = [(lanes & s) == 0 for s in (1, 2, 4, 8)]

        def transpose16(vv):
            for si, s in enumerate((1, 2, 4, 8)):
                nv = []
                for i in range(LANES):
                    p = vv[i ^ s].at[perms[si]].get(mode="promise_in_bounds")
                    if i & s == 0:
                        nv.append(jnp.where(masks[si], vv[i], p))
                    else:
                        nv.append(jnp.where(masks[si], p, vv[i]))
                vv = nv
            return vv

        def in_view(k):
            return table_hbm.at[:, pl.ds((start + k * CH) * 128, CH * 128)]

        def out_rows(k):
            return out_hbm.at[pl.ds((start + k * CH) * dim, CH * dim), :]

        def phase_full(k, b):
            pltpu.make_async_copy(in_view(k), vbuf.at[b], isem.at[b]).wait()

            @pl.when(k >= NBUF_F)
            def _drain_out():
                pltpu.make_async_copy(slab.at[b], out_rows(k - NBUF_F),
                                      osem.at[b]).wait()

            for cc in range(CH):
                for g in range(8):
                    vv = [vbuf[b, r, pl.ds(cc * 128 + g * LANES, LANES)]
                          for r in range(LANES)]
                    vv = transpose16(vv)
                    for j in range(8):
                        slab[b, cc * LANES + 2 * g, pl.ds(j * LANES, LANES)] = vv[j]
                        slab[b, cc * LANES + 2 * g + 1, pl.ds(j * LANES, LANES)] = vv[8 + j]
            pltpu.async_copy(slab.at[b], out_rows(k), osem.at[b])

            @pl.when(k + NBUF_F < n_chunks)
            def _next_in():
                pltpu.async_copy(in_view(k + NBUF_F), vbuf.at[b], isem.at[b])

        for b in range(NBUF_F):
            pltpu.async_copy(in_view(b), vbuf.at[b], isem.at[b])

        def body(k, _):
            for b in range(NBUF_F):
                @pl.when(k % NBUF_F == b)
                def _ph(b=b):
                    phase_full(k, b)
            return _

        lax.fori_loop(0, n_chunks, body, None)
        for k in range(n_chunks - NBUF_F, n_chunks):
            pltpu.make_async_copy(slab.at[k % NBUF_F], out_rows(k),
                                  osem.at[k % NBUF_F]).wait()

        @pl.when(wid < n_extra)
        def _extra():
            c = start + base_w
            pltpu.sync_copy(table_hbm.at[:, pl.ds(c * 128, 128)],
                            vbuf.at[0, :, pl.ds(0, 128)])
            for g in range(8):
                vv = [vbuf[0, r, pl.ds(g * LANES, LANES)]
                      for r in range(LANES)]
                vv = transpose16(vv)
                for j in range(8):
                    slab[0, 2 * g, pl.ds(j * LANES, LANES)] = vv[j]
                    slab[0, 2 * g + 1, pl.ds(j * LANES, LANES)] = vv[8 + j]
            pltpu.sync_copy(slab.at[0, pl.ds(0, dim), :],
                            out_hbm.at[pl.ds(c * dim, dim), :])

        if rem:
            @pl.when(wid == NUM_WORKERS - 1)
            def _tail():
                pltpu.sync_copy(rem_hbm,
                                vbuf.at[0, pl.ds(0, rem_rows), pl.ds(0, 128)])
                pltpu.sync_copy(vbuf.at[0, pl.ds(0, rem_rows), pl.ds(0, 128)],
                                out_hbm.at[pl.ds(full_tcols * dim, rem_rows), :])

    return format_kernel


@functools.lru_cache(maxsize=None)
def _build_gather(batch: int, fields: int, vocab: int, dim: int):
    per_w = batch // NUM_WORKERS
    n_blocks = per_w // BLOCK
    assert batch % NUM_WORKERS == 0 and per_w % BLOCK == 0 and dim == LANES
    groups = BLOCK // LANES
    n_tcols = batch // 128
    mesh = plsc.VectorSubcoreMesh(core_axis_name="c", subcore_axis_name="s")

    @functools.partial(
        pl.kernel,
        mesh=mesh,
        compiler_params=pltpu.CompilerParams(use_tc_tiling_on_sc=False),
        out_type=jax.ShapeDtypeStruct((fields, dim // 8, n_tcols, 8, 128),
                                      jnp.float32),
        scratch_types=[
            pltpu.VMEM((fields, per_w), jnp.int32),
            pltpu.VMEM((2, fields * BLOCK, dim), jnp.float32),
            pltpu.VMEM((2, fields, dim // 8, 8, BLOCK), jnp.float32),
            pltpu.SemaphoreType.DMA((2,)),
            pltpu.SemaphoreType.DMA((2,)),
        ],
    )
    def gather_kernel(table_hbm, idx_hbm, out_hbm, idx_v, rows_v, slab_v,
                      gsem, osem):
        wid = lax.axis_index("s") * NUM_CORES + lax.axis_index("c")
        base = wid * per_w
        pltpu.sync_copy(idx_hbm.at[:, pl.ds(base, per_w)], idx_v)
        lanes = lax.iota(jnp.int32, LANES)
        perms = [lanes ^ s for s in (1, 2, 4, 8)]
        masks = [(lanes & s) == 0 for s in (1, 2, 4, 8)]

        def transpose16(vv):
            for si, s in enumerate((1, 2, 4, 8)):
                nv = []
                for i in range(LANES):
                    p = vv[i ^ s].at[perms[si]].get(mode="promise_in_bounds")
                    if i & s == 0:
                        nv.append(jnp.where(masks[si], vv[i], p))
                    else:
                        nv.append(jnp.where(masks[si], p, vv[i]))
                vv = nv
            return vv

        def gather_start(k):
            b = k % 2
            for f in range(fields):
                pltpu.async_copy(
                    table_hbm.at[idx_v.at[f, pl.ds(k * BLOCK, BLOCK)]],
                    rows_v.at[b, pl.ds(f * BLOCK, BLOCK)], gsem.at[b])

        def gather_drain(k):
            b = k % 2
            for f in range(fields):
                pltpu.make_async_copy(
                    table_hbm.at[idx_v.at[f, pl.ds(k * BLOCK, BLOCK)]],
                    rows_v.at[b, pl.ds(f * BLOCK, BLOCK)], gsem.at[b]).wait()

        def out_view(k):
            b0 = base + k * BLOCK
            return out_hbm.at[:, :, b0 // 128, :, pl.ds(b0 % 128, BLOCK)]

        def out_start(k):
            b = k % 2
            pltpu.async_copy(slab_v.at[b], out_view(k), osem.at[b])

        def out_drain(k):
            b = k % 2
            pltpu.make_async_copy(slab_v.at[b], out_view(k), osem.at[b]).wait()

        def transpose_block(k):
            b = k % 2

            def tp_body(t, _):
                f = t // groups
                g = t % groups
                rbase = f * BLOCK + g * LANES
                vv = [rows_v[b, rbase + r, :] for r in range(LANES)]
                vv = transpose16(vv)
                for d in range(LANES):
                    slab_v[b, f, d // 8, d % 8, pl.ds(g * LANES, LANES)] = vv[d]
                return _

            lax.fori_loop(0, fields * groups, tp_body, None)

        gather_start(0)
        for k in range(n_blocks):
            if k + 1 < n_blocks:
                gather_start(k + 1)
            gather_drain(k)
            if k >= 2:
                out_drain(k - 2)
            transpose_block(k)
            out_start(k)
        for k in range(max(0, n_blocks - 2), n_blocks):
            out_drain(k)

    return gather_kernel


def kernel(inputs, embed_weight):
    batch, fields = inputs.shape
    vocab, dim = embed_weight.shape
    idx_t = inputs.T.astype(jnp.int32)
    fmt = _build_format(vocab, dim)
    full_tcols = vocab // 128
    rem8 = embed_weight[full_tcols * 128:, :].reshape(-1, 128)
    table_lin = fmt(embed_weight.T, rem8).reshape(vocab, dim)
    gather = _build_gather(batch, fields, vocab, dim)
    out5 = gather(table_lin, idx_t)
    out = out5.transpose(2, 4, 0, 1, 3).reshape(batch, fields, dim)
    return out
